# Initial kernel scaffold; baseline (speedup 1.0000x reference)
import jax
import jax.numpy as jnp
from jax.experimental import pallas as pl


def kernel(tensor_a, tensor_b, theta1_intra, theta2_intra, theta_a_inter, theta_b_inter):
    raise NotImplementedError("write your pallas kernel here")


# speedup vs baseline: 9.6809x; 9.6809x over previous
import functools

import jax
import jax.numpy as jnp
from jax.experimental import pallas as pl

_B, _N, _D = 8, 512, 256
_K = 8


def _topk_keep(P):
    iota = jax.lax.broadcasted_iota(jnp.int32, P.shape, 1)
    work = P
    keep = jnp.zeros(P.shape, jnp.bool_)
    for _ in range(_K):
        m = jnp.max(work, axis=1, keepdims=True)
        ismax = work == m
        idx = jnp.min(jnp.where(ismax, iota, _N), axis=1, keepdims=True)
        sel = iota == idx
        keep = keep | sel
        work = jnp.where(sel, -jnp.inf, work)
    return jnp.where(keep, P, 0.0)


def _dotT(x, y):
    return jax.lax.dot_general(
        x, y, dimension_numbers=(((1,), (1,)), ((), ())),
        preferred_element_type=jnp.float32)


def _body(a_ref, b_ref, t1_ref, t2_ref, ta_ref, tb_ref,
          oa_ref, ob_ref, oi_ref):
    sig = jax.nn.sigmoid
    a = a_ref[0]
    b = b_ref[0]
    t1 = t1_ref[...]
    t2 = t2_ref[...]
    ta = ta_ref[...]
    tb = tb_ref[...]

    ua1 = a * t1
    ua2 = a * t2
    oa_ref[0] = _topk_keep(jax.nn.relu(sig(_dotT(ua1, ua2)) - sig(_dotT(ua2, ua1))))

    ub1 = b * t1
    ub2 = b * t2
    ob_ref[0] = _topk_keep(jax.nn.relu(sig(_dotT(ub1, ub2)) - sig(_dotT(ub2, ub1))))

    ua = a * ta
    ub = b * tb
    oi_ref[0] = _topk_keep(jax.nn.relu(sig(_dotT(ua, ub))))


@functools.partial(jax.jit, static_argnames=())
def kernel(tensor_a, tensor_b, theta1_intra, theta2_intra,
           theta_a_inter, theta_b_inter):
    t1 = theta1_intra.reshape(1, _D)
    t2 = theta2_intra.reshape(1, _D)
    ta = theta_a_inter.reshape(1, _D)
    tb = theta_b_inter.reshape(1, _D)

    batch_spec = pl.BlockSpec((1, _N, _D), lambda i: (i, 0, 0))
    theta_spec = pl.BlockSpec((1, _D), lambda i: (0, 0))
    out_spec = pl.BlockSpec((1, _N, _N), lambda i: (i, 0, 0))
    out_shape = jax.ShapeDtypeStruct((_B, _N, _N), jnp.float32)

    return pl.pallas_call(
        _body,
        grid=(_B,),
        in_specs=[batch_spec, batch_spec,
                  theta_spec, theta_spec, theta_spec, theta_spec],
        out_specs=[out_spec, out_spec, out_spec],
        out_shape=[out_shape, out_shape, out_shape],
    )(tensor_a, tensor_b, t1, t2, ta, tb)
